# baseline (speedup 1.0000x reference)
import functools

import jax
import jax.numpy as jnp
from jax import lax
from jax.experimental import pallas as pl
from jax.experimental.pallas import tpu as pltpu
from jax.experimental.pallas import tpu_sc as plsc

_CH = 128
_NBUF = 4
_BLK = 1024


def _sc_info():
    try:
        info = plsc.get_sparse_core_info()
        return info.num_cores, info.num_subcores
    except Exception:
        return 2, 16


@functools.lru_cache(maxsize=None)
def _make_deg(npad, nch, nc, ns):
    mesh = plsc.VectorSubcoreMesh(core_axis_name="c", subcore_axis_name="s",
                                  num_cores=nc, num_subcores=ns)
    rows_per_tile = npad // ns

    def body(dst_hbm, zero_hbm, out_hbm, dstv, ones_v, acc):
        c = lax.axis_index("c")
        s = lax.axis_index("s")
        wid = s * nc + c
        pltpu.sync_copy(dst_hbm.at[wid], dstv)
        for i in range(_CH // 16):
            ones_v[pl.ds(i * 16, 16)] = jnp.full((16,), 1.0, jnp.float32)

        @pl.when(s == 0)
        def _():
            pltpu.sync_copy(zero_hbm, acc)

        plsc.subcore_barrier()

        def step(a, carry):
            pltpu.sync_copy(ones_v, acc.at[dstv.at[a]], add=True)
            return carry

        lax.fori_loop(0, nch, step, 0)
        plsc.subcore_barrier()
        lo = s * rows_per_tile
        pltpu.sync_copy(acc.at[pl.ds(lo, rows_per_tile)],
                        out_hbm.at[c].at[pl.ds(lo, rows_per_tile)])

    return pl.kernel(
        body,
        out_type=jax.ShapeDtypeStruct((nc, npad), jnp.float32),
        mesh=mesh,
        compiler_params=pltpu.CompilerParams(use_tc_tiling_on_sc=False),
        scratch_types=[
            pltpu.VMEM((nch, _CH), jnp.int32),
            pltpu.VMEM((_CH,), jnp.float32),
            pltpu.VMEM_SHARED((npad,), jnp.float32),
        ],
    )


@functools.lru_cache(maxsize=None)
def _make_spmm(npad, d, nch, nc, ns):
    mesh = plsc.VectorSubcoreMesh(core_axis_name="c", subcore_axis_name="s",
                                  num_cores=nc, num_subcores=ns)
    rows_per_tile = npad // ns

    nbuf = _NBUF

    def body(y_hbm, src_hbm, dst_hbm, zero_hbm, out_hbm,
             srcv, dstv, rows, gsems, ssems, acc):
        c = lax.axis_index("c")
        s = lax.axis_index("s")
        wid = s * nc + c
        pltpu.sync_copy(src_hbm.at[wid], srcv)
        pltpu.sync_copy(dst_hbm.at[wid], dstv)

        @pl.when(s == 0)
        def _():
            pltpu.sync_copy(zero_hbm, acc)

        plsc.subcore_barrier()

        for b in range(nbuf):
            pltpu.async_copy(y_hbm.at[srcv.at[b]], rows[b], gsems[b])

        def outer(g2, carry):
            g = g2 * nbuf
            for b in range(nbuf):
                a = g + b
                pltpu.make_async_copy(y_hbm.at[srcv.at[a]], rows[b],
                                      gsems[b]).wait()
                pltpu.async_copy(rows[b], acc.at[dstv.at[a]], ssems[b],
                                 add=True)
            for b in range(nbuf):
                a = g + b
                pltpu.make_async_copy(rows[b], acc.at[dstv.at[a]],
                                      ssems[b]).wait()

                @pl.when(a + nbuf < nch)
                def _():
                    pltpu.async_copy(y_hbm.at[srcv.at[a + nbuf]], rows[b],
                                     gsems[b])
            return carry

        lax.fori_loop(0, nch // nbuf, outer, 0)
        plsc.subcore_barrier()
        lo = s * rows_per_tile
        pltpu.sync_copy(acc.at[pl.ds(lo, rows_per_tile)],
                        out_hbm.at[c].at[pl.ds(lo, rows_per_tile)])

    return pl.kernel(
        body,
        out_type=jax.ShapeDtypeStruct((nc, npad, d), jnp.float32),
        mesh=mesh,
        compiler_params=pltpu.CompilerParams(use_tc_tiling_on_sc=False),
        scratch_types=[
            pltpu.VMEM((nch, _CH), jnp.int32),
            pltpu.VMEM((nch, _CH), jnp.int32),
            [pltpu.VMEM((_CH, d), jnp.float32) for _ in range(nbuf)],
            [pltpu.SemaphoreType.DMA for _ in range(nbuf)],
            [pltpu.SemaphoreType.DMA for _ in range(nbuf)],
            pltpu.VMEM_SHARED((npad, d), jnp.float32),
        ],
    )


def _tc0_body(degp, xb, f0b, w0, wl0, bl0, y0, lin0, dinvb):
    deg = degp[0, :] + degp[1, :]
    dinv = lax.rsqrt(jnp.maximum(deg, 1.0))[:, None]
    dinvb[...] = dinv
    y0[...] = jnp.dot(xb[...], w0[...], preferred_element_type=jnp.float32) * dinv
    lin0[...] = jnp.maximum(
        jnp.dot(f0b[...], wl0[...], preferred_element_type=jnp.float32) + bl0[...], 0.0)


def _tc1_body(p0, lin0, f1b, f2b, dinvb, w1, wl1a, wl1b, b0, bl1, y1, lin1):
    dinv = dinvb[...]
    conv0 = jnp.maximum(dinv * (p0[0] + p0[1]) + b0[...], 0.0)
    comb = jnp.concatenate([lin0[...], conv0], axis=1)
    y1[...] = jnp.dot(comb, w1[...], preferred_element_type=jnp.float32) * dinv
    lin1[...] = jnp.maximum(
        jnp.dot(f1b[...], wl1a[...], preferred_element_type=jnp.float32)
        + jnp.dot(f2b[...], wl1b[...], preferred_element_type=jnp.float32)
        + bl1[...], 0.0)


def _tc2_body(p1, lin1, dinvb, wc, b1, yc):
    dinv = dinvb[...]
    conv1 = jnp.maximum(dinv * (p1[0] + p1[1]) + b1[...], 0.0)
    comb = jnp.concatenate([lin1[...], conv1], axis=1)
    yc[...] = jnp.dot(comb, wc[...], preferred_element_type=jnp.float32) * dinv


def _tc3_body(pc, dinvb, bc, out0, convc):
    v = dinvb[...] * (pc[0] + pc[1]) + bc[...]
    m = jnp.max(v, axis=-1, keepdims=True)
    lse = jnp.log(jnp.sum(jnp.exp(v - m), axis=-1, keepdims=True)) + m
    convc[...] = v
    out0[...] = v - lse


def _row_spec(d):
    return pl.BlockSpec((_BLK, d), lambda i: (i, 0))


def _pair_spec(d):
    return pl.BlockSpec((2, _BLK, d), lambda i: (0, i, 0))


def _full_spec(shape):
    return pl.BlockSpec(shape, lambda i: tuple(0 for _ in shape))


def kernel(x, edge_index, features, W0, b0, Wl0, bl0, W1, b1, Wl1, bl1,
           order_weights, Wc, bc):
    n, fin = x.shape
    nhid = W0.shape[1]
    ncls = Wc.shape[1]
    nc, ns = _sc_info()
    nw = nc * ns

    npad = ((n + 1 + _BLK - 1) // _BLK) * _BLK
    grid = npad // _BLK

    idt = edge_index.dtype
    loop = jnp.arange(n, dtype=idt)
    src = jnp.concatenate([edge_index[0], loop])
    dst = jnp.concatenate([edge_index[1], loop])
    et = src.shape[0]
    nch = -(-et // (nw * _CH))
    nch = -(-nch // _NBUF) * _NBUF
    epad = nw * nch * _CH
    srcp = jnp.full((epad,), n, idt).at[:et].set(src).reshape(nw, nch, _CH)
    dstp = jnp.full((epad,), n, idt).at[:et].set(dst).reshape(nw, nch, _CH)

    xp = jnp.zeros((npad, fin), jnp.float32).at[:n].set(x)
    f0p = jnp.zeros((npad, fin), jnp.float32).at[:n].set(features[0])
    f1p = jnp.zeros((npad, fin), jnp.float32).at[:n].set(features[1])
    f2p = jnp.zeros((npad, fin), jnp.float32).at[:n].set(features[2])

    wl1a = order_weights[0] * Wl1[:fin]
    wl1b = order_weights[1] * Wl1[fin:]
    b0r = b0[None, :]
    b1r = b1[None, :]
    bl0r = bl0[None, :]
    bl1r = bl1[None, :]
    bcr = bc[None, :]

    zdeg = jnp.zeros((npad,), jnp.float32)
    zh = jnp.zeros((npad, nhid), jnp.float32)
    zc = jnp.zeros((npad, ncls), jnp.float32)

    deg_fn = _make_deg(npad, nch, nc, ns)
    spmm_h = _make_spmm(npad, nhid, nch, nc, ns)
    spmm_c = _make_spmm(npad, ncls, nch, nc, ns)

    degp = deg_fn(dstp, zdeg)

    y0, lin0, dinv = pl.pallas_call(
        _tc0_body,
        grid=(grid,),
        in_specs=[
            pl.BlockSpec((2, _BLK), lambda i: (0, i)),
            _row_spec(fin), _row_spec(fin),
            _full_spec((fin, nhid)), _full_spec((fin, nhid)),
            _full_spec((1, nhid)),
        ],
        out_specs=[_row_spec(nhid), _row_spec(nhid), _row_spec(1)],
        out_shape=[
            jax.ShapeDtypeStruct((npad, nhid), jnp.float32),
            jax.ShapeDtypeStruct((npad, nhid), jnp.float32),
            jax.ShapeDtypeStruct((npad, 1), jnp.float32),
        ],
    )(degp, xp, f0p, W0, Wl0, bl0r)

    p0 = spmm_h(y0, srcp, dstp, zh)

    y1, lin1 = pl.pallas_call(
        _tc1_body,
        grid=(grid,),
        in_specs=[
            _pair_spec(nhid), _row_spec(nhid), _row_spec(fin), _row_spec(fin),
            _row_spec(1),
            _full_spec((fin, nhid)), _full_spec((fin, nhid)),
            _full_spec((fin, nhid)),
            _full_spec((1, nhid)), _full_spec((1, nhid)),
        ],
        out_specs=[_row_spec(nhid), _row_spec(nhid)],
        out_shape=[
            jax.ShapeDtypeStruct((npad, nhid), jnp.float32),
            jax.ShapeDtypeStruct((npad, nhid), jnp.float32),
        ],
    )(p0, lin0, f1p, f2p, dinv, W1, wl1a, wl1b, b0r, bl1r)

    p1 = spmm_h(y1, srcp, dstp, zh)

    yc = pl.pallas_call(
        _tc2_body,
        grid=(grid,),
        in_specs=[
            _pair_spec(nhid), _row_spec(nhid), _row_spec(1),
            _full_spec((2 * nhid, ncls)), _full_spec((1, nhid)),
        ],
        out_specs=_row_spec(ncls),
        out_shape=jax.ShapeDtypeStruct((npad, ncls), jnp.float32),
    )(p1, lin1, dinv, Wc, b1r)

    pc = spmm_c(yc, srcp, dstp, zc)

    out0, convc = pl.pallas_call(
        _tc3_body,
        grid=(grid,),
        in_specs=[_pair_spec(ncls), _row_spec(1), _full_spec((1, ncls))],
        out_specs=[_row_spec(ncls), _row_spec(ncls)],
        out_shape=[
            jax.ShapeDtypeStruct((npad, ncls), jnp.float32),
            jax.ShapeDtypeStruct((npad, ncls), jnp.float32),
        ],
    )(pc, dinv, bcr)

    return (out0[:n], convc[:n])


# speedup vs baseline: 1.0155x; 1.0155x over previous
import functools

import jax
import jax.numpy as jnp
from jax import lax
from jax.experimental import pallas as pl
from jax.experimental.pallas import tpu as pltpu
from jax.experimental.pallas import tpu_sc as plsc

_CH = 128
_NBUF = 4
_BLK = 1024


def _sc_info():
    try:
        info = plsc.get_sparse_core_info()
        return info.num_cores, info.num_subcores
    except Exception:
        return 2, 16


@functools.lru_cache(maxsize=None)
def _make_deg(npad, nch, nc, ns):
    mesh = plsc.VectorSubcoreMesh(core_axis_name="c", subcore_axis_name="s",
                                  num_cores=nc, num_subcores=ns)
    rows_per_tile = npad // ns

    def body(dst_hbm, zero_hbm, out_hbm, dstv, ones_v, acc):
        c = lax.axis_index("c")
        s = lax.axis_index("s")
        wid = s * nc + c
        pltpu.sync_copy(dst_hbm.at[wid], dstv)
        for i in range(_CH // 16):
            ones_v[pl.ds(i * 16, 16)] = jnp.full((16,), 1.0, jnp.float32)

        @pl.when(s == 0)
        def _():
            pltpu.sync_copy(zero_hbm, acc)

        plsc.subcore_barrier()

        def step(a, carry):
            pltpu.sync_copy(ones_v, acc.at[dstv.at[a]], add=True)
            return carry

        lax.fori_loop(0, nch, step, 0)
        plsc.subcore_barrier()
        lo = s * rows_per_tile
        pltpu.sync_copy(acc.at[pl.ds(lo, rows_per_tile)],
                        out_hbm.at[c].at[pl.ds(lo, rows_per_tile)])

    return pl.kernel(
        body,
        out_type=jax.ShapeDtypeStruct((nc, npad), jnp.float32),
        mesh=mesh,
        compiler_params=pltpu.CompilerParams(use_tc_tiling_on_sc=False),
        scratch_types=[
            pltpu.VMEM((nch, _CH), jnp.int32),
            pltpu.VMEM((_CH,), jnp.float32),
            pltpu.VMEM_SHARED((npad,), jnp.float32),
        ],
    )


@functools.lru_cache(maxsize=None)
def _make_spmm(npad, d, nch, nc, ns):
    mesh = plsc.VectorSubcoreMesh(core_axis_name="c", subcore_axis_name="s",
                                  num_cores=nc, num_subcores=ns)
    rows_per_tile = npad // ns

    nbuf = _NBUF

    def body(y_hbm, src_hbm, dst_hbm, zero_hbm, out_hbm,
             srcv, dstv, rows, gsems, acc):
        c = lax.axis_index("c")
        s = lax.axis_index("s")
        wid = s * nc + c
        pltpu.sync_copy(src_hbm.at[wid], srcv)
        pltpu.sync_copy(dst_hbm.at[wid], dstv)

        @pl.when(s == 0)
        def _():
            pltpu.sync_copy(zero_hbm, acc)

        plsc.subcore_barrier()

        for b in range(nbuf - 1):
            pltpu.async_copy(y_hbm.at[srcv.at[b]], rows[b], gsems[b])

        def outer(g2, carry):
            g = g2 * nbuf
            for b in range(nbuf):
                a = g + b
                pltpu.make_async_copy(y_hbm.at[srcv.at[a]], rows[b],
                                      gsems[b]).wait()

                nb = (b + nbuf - 1) % nbuf

                @pl.when(a + nbuf - 1 < nch)
                def _():
                    pltpu.async_copy(y_hbm.at[srcv.at[a + nbuf - 1]],
                                     rows[nb], gsems[nb])

                pltpu.sync_copy(rows[b], acc.at[dstv.at[a]], add=True)
            return carry

        lax.fori_loop(0, nch // nbuf, outer, 0)
        plsc.subcore_barrier()
        lo = s * rows_per_tile
        pltpu.sync_copy(acc.at[pl.ds(lo, rows_per_tile)],
                        out_hbm.at[c].at[pl.ds(lo, rows_per_tile)])

    return pl.kernel(
        body,
        out_type=jax.ShapeDtypeStruct((nc, npad, d), jnp.float32),
        mesh=mesh,
        compiler_params=pltpu.CompilerParams(use_tc_tiling_on_sc=False),
        scratch_types=[
            pltpu.VMEM((nch, _CH), jnp.int32),
            pltpu.VMEM((nch, _CH), jnp.int32),
            [pltpu.VMEM((_CH, d), jnp.float32) for _ in range(nbuf)],
            [pltpu.SemaphoreType.DMA for _ in range(nbuf)],
            pltpu.VMEM_SHARED((npad, d), jnp.float32),
        ],
    )


def _tc0_body(degp, xb, f0b, w0, wl0, bl0, y0, lin0, dinvb):
    deg = degp[0, :] + degp[1, :]
    dinv = lax.rsqrt(jnp.maximum(deg, 1.0))[:, None]
    dinvb[...] = dinv
    y0[...] = jnp.dot(xb[...], w0[...], preferred_element_type=jnp.float32) * dinv
    lin0[...] = jnp.maximum(
        jnp.dot(f0b[...], wl0[...], preferred_element_type=jnp.float32) + bl0[...], 0.0)


def _tc1_body(p0, lin0, f1b, f2b, dinvb, w1, wl1a, wl1b, b0, bl1, y1, lin1):
    dinv = dinvb[...]
    conv0 = jnp.maximum(dinv * (p0[0] + p0[1]) + b0[...], 0.0)
    comb = jnp.concatenate([lin0[...], conv0], axis=1)
    y1[...] = jnp.dot(comb, w1[...], preferred_element_type=jnp.float32) * dinv
    lin1[...] = jnp.maximum(
        jnp.dot(f1b[...], wl1a[...], preferred_element_type=jnp.float32)
        + jnp.dot(f2b[...], wl1b[...], preferred_element_type=jnp.float32)
        + bl1[...], 0.0)


def _tc2_body(p1, lin1, dinvb, wc, b1, yc):
    dinv = dinvb[...]
    conv1 = jnp.maximum(dinv * (p1[0] + p1[1]) + b1[...], 0.0)
    comb = jnp.concatenate([lin1[...], conv1], axis=1)
    yc[...] = jnp.dot(comb, wc[...], preferred_element_type=jnp.float32) * dinv


def _tc3_body(pc, dinvb, bc, out0, convc):
    v = dinvb[...] * (pc[0] + pc[1]) + bc[...]
    m = jnp.max(v, axis=-1, keepdims=True)
    lse = jnp.log(jnp.sum(jnp.exp(v - m), axis=-1, keepdims=True)) + m
    convc[...] = v
    out0[...] = v - lse


def _row_spec(d):
    return pl.BlockSpec((_BLK, d), lambda i: (i, 0))


def _pair_spec(d):
    return pl.BlockSpec((2, _BLK, d), lambda i: (0, i, 0))


def _full_spec(shape):
    return pl.BlockSpec(shape, lambda i: tuple(0 for _ in shape))


def kernel(x, edge_index, features, W0, b0, Wl0, bl0, W1, b1, Wl1, bl1,
           order_weights, Wc, bc):
    n, fin = x.shape
    nhid = W0.shape[1]
    ncls = Wc.shape[1]
    nc, ns = _sc_info()
    nw = nc * ns

    npad = ((n + 1 + _BLK - 1) // _BLK) * _BLK
    grid = npad // _BLK

    idt = edge_index.dtype
    loop = jnp.arange(n, dtype=idt)
    src = jnp.concatenate([edge_index[0], loop])
    dst = jnp.concatenate([edge_index[1], loop])
    et = src.shape[0]
    nch = -(-et // (nw * _CH))
    nch = -(-nch // _NBUF) * _NBUF
    epad = nw * nch * _CH
    srcp = jnp.full((epad,), n, idt).at[:et].set(src).reshape(nw, nch, _CH)
    dstp = jnp.full((epad,), n, idt).at[:et].set(dst).reshape(nw, nch, _CH)

    xp = jnp.zeros((npad, fin), jnp.float32).at[:n].set(x)
    f0p = jnp.zeros((npad, fin), jnp.float32).at[:n].set(features[0])
    f1p = jnp.zeros((npad, fin), jnp.float32).at[:n].set(features[1])
    f2p = jnp.zeros((npad, fin), jnp.float32).at[:n].set(features[2])

    wl1a = order_weights[0] * Wl1[:fin]
    wl1b = order_weights[1] * Wl1[fin:]
    b0r = b0[None, :]
    b1r = b1[None, :]
    bl0r = bl0[None, :]
    bl1r = bl1[None, :]
    bcr = bc[None, :]

    zdeg = jnp.zeros((npad,), jnp.float32)
    zh = jnp.zeros((npad, nhid), jnp.float32)
    zc = jnp.zeros((npad, ncls), jnp.float32)

    deg_fn = _make_deg(npad, nch, nc, ns)
    spmm_h = _make_spmm(npad, nhid, nch, nc, ns)
    spmm_c = _make_spmm(npad, ncls, nch, nc, ns)

    degp = deg_fn(dstp, zdeg)

    y0, lin0, dinv = pl.pallas_call(
        _tc0_body,
        grid=(grid,),
        in_specs=[
            pl.BlockSpec((2, _BLK), lambda i: (0, i)),
            _row_spec(fin), _row_spec(fin),
            _full_spec((fin, nhid)), _full_spec((fin, nhid)),
            _full_spec((1, nhid)),
        ],
        out_specs=[_row_spec(nhid), _row_spec(nhid), _row_spec(1)],
        out_shape=[
            jax.ShapeDtypeStruct((npad, nhid), jnp.float32),
            jax.ShapeDtypeStruct((npad, nhid), jnp.float32),
            jax.ShapeDtypeStruct((npad, 1), jnp.float32),
        ],
    )(degp, xp, f0p, W0, Wl0, bl0r)

    p0 = spmm_h(y0, srcp, dstp, zh)

    y1, lin1 = pl.pallas_call(
        _tc1_body,
        grid=(grid,),
        in_specs=[
            _pair_spec(nhid), _row_spec(nhid), _row_spec(fin), _row_spec(fin),
            _row_spec(1),
            _full_spec((fin, nhid)), _full_spec((fin, nhid)),
            _full_spec((fin, nhid)),
            _full_spec((1, nhid)), _full_spec((1, nhid)),
        ],
        out_specs=[_row_spec(nhid), _row_spec(nhid)],
        out_shape=[
            jax.ShapeDtypeStruct((npad, nhid), jnp.float32),
            jax.ShapeDtypeStruct((npad, nhid), jnp.float32),
        ],
    )(p0, lin0, f1p, f2p, dinv, W1, wl1a, wl1b, b0r, bl1r)

    p1 = spmm_h(y1, srcp, dstp, zh)

    yc = pl.pallas_call(
        _tc2_body,
        grid=(grid,),
        in_specs=[
            _pair_spec(nhid), _row_spec(nhid), _row_spec(1),
            _full_spec((2 * nhid, ncls)), _full_spec((1, nhid)),
        ],
        out_specs=_row_spec(ncls),
        out_shape=jax.ShapeDtypeStruct((npad, ncls), jnp.float32),
    )(p1, lin1, dinv, Wc, b1r)

    pc = spmm_c(yc, srcp, dstp, zc)

    out0, convc = pl.pallas_call(
        _tc3_body,
        grid=(grid,),
        in_specs=[_pair_spec(ncls), _row_spec(1), _full_spec((1, ncls))],
        out_specs=[_row_spec(ncls), _row_spec(ncls)],
        out_shape=[
            jax.ShapeDtypeStruct((npad, ncls), jnp.float32),
            jax.ShapeDtypeStruct((npad, ncls), jnp.float32),
        ],
    )(pc, dinv, bcr)

    return (out0[:n], convc[:n])


# speedup vs baseline: 1.4686x; 1.4462x over previous
import functools

import jax
import jax.numpy as jnp
from jax import lax
from jax.experimental import pallas as pl
from jax.experimental.pallas import tpu as pltpu
from jax.experimental.pallas import tpu_sc as plsc

_CH = 128
_NBUF = 2
_BLK = 1024


def _sc_info():
    try:
        info = plsc.get_sparse_core_info()
        return info.num_cores, info.num_subcores
    except Exception:
        return 2, 16


@functools.lru_cache(maxsize=None)
def _make_deg(npad, nch, nc, ns):
    mesh = plsc.VectorSubcoreMesh(core_axis_name="c", subcore_axis_name="s",
                                  num_cores=nc, num_subcores=ns)
    rows_per_tile = npad // ns

    def body(dst_hbm, zero_hbm, out_hbm, dstv, ones_v, acc):
        c = lax.axis_index("c")
        s = lax.axis_index("s")
        wid = s * nc + c
        pltpu.sync_copy(dst_hbm.at[wid], dstv)
        for i in range(_CH // 16):
            ones_v[pl.ds(i * 16, 16)] = jnp.full((16,), 1.0, jnp.float32)

        @pl.when(s == 0)
        def _():
            pltpu.sync_copy(zero_hbm, acc)

        plsc.subcore_barrier()

        def step(a, carry):
            pltpu.sync_copy(ones_v, acc.at[dstv.at[a]], add=True)
            return carry

        lax.fori_loop(0, nch, step, 0)
        plsc.subcore_barrier()
        lo = s * rows_per_tile
        pltpu.sync_copy(acc.at[pl.ds(lo, rows_per_tile)],
                        out_hbm.at[c].at[pl.ds(lo, rows_per_tile)])

    return pl.kernel(
        body,
        out_type=jax.ShapeDtypeStruct((nc, npad), jnp.float32),
        mesh=mesh,
        compiler_params=pltpu.CompilerParams(use_tc_tiling_on_sc=False),
        scratch_types=[
            pltpu.VMEM((nch, _CH), jnp.int32),
            pltpu.VMEM((_CH,), jnp.float32),
            pltpu.VMEM_SHARED((npad,), jnp.float32),
        ],
    )


@functools.lru_cache(maxsize=None)
def _make_spmm(npad, d, nch, nc, ns):
    mesh = plsc.VectorSubcoreMesh(core_axis_name="c", subcore_axis_name="s",
                                  num_cores=nc, num_subcores=ns)
    rows_per_tile = npad // ns

    nbuf = _NBUF

    def body(y_hbm, src_hbm, dst_hbm, zero_hbm, out_hbm,
             srcv, dstv, rows, gsems, acc):
        c = lax.axis_index("c")
        s = lax.axis_index("s")
        wid = s * nc + c
        pltpu.sync_copy(src_hbm.at[wid], srcv)
        pltpu.sync_copy(dst_hbm.at[wid], dstv)

        @pl.when(s == 0)
        def _():
            pltpu.sync_copy(zero_hbm, acc)

        plsc.subcore_barrier()

        for b in range(nbuf - 1):
            pltpu.async_copy(y_hbm.at[srcv.at[b]], rows[b], gsems[b])

        def outer(g2, carry):
            g = g2 * nbuf
            for b in range(nbuf):
                a = g + b
                pltpu.make_async_copy(y_hbm.at[srcv.at[a]], rows[b],
                                      gsems[b]).wait()

                nb = (b + nbuf - 1) % nbuf

                @pl.when(a + nbuf - 1 < nch)
                def _():
                    pltpu.async_copy(y_hbm.at[srcv.at[a + nbuf - 1]],
                                     rows[nb], gsems[nb])

                pltpu.sync_copy(rows[b], acc.at[dstv.at[a]], add=True)
            return carry

        lax.fori_loop(0, nch // nbuf, outer, 0)
        plsc.subcore_barrier()
        lo = s * rows_per_tile
        pltpu.sync_copy(acc.at[pl.ds(lo, rows_per_tile)],
                        out_hbm.at[c].at[pl.ds(lo, rows_per_tile)])

    return pl.kernel(
        body,
        out_type=jax.ShapeDtypeStruct((nc, npad, d), jnp.float32),
        mesh=mesh,
        compiler_params=pltpu.CompilerParams(use_tc_tiling_on_sc=False),
        scratch_types=[
            pltpu.VMEM((nch, _CH), jnp.int32),
            pltpu.VMEM((nch, _CH), jnp.int32),
            [pltpu.VMEM((_CH, d), jnp.float32) for _ in range(nbuf)],
            [pltpu.SemaphoreType.DMA for _ in range(nbuf)],
            pltpu.VMEM_SHARED((npad, d), jnp.float32),
        ],
    )


def _tc0_body(degp, xb, f0b, w0, wl0, bl0, y0, lin0, dinvb):
    deg = degp[0, :] + degp[1, :]
    dinv = lax.rsqrt(jnp.maximum(deg, 1.0))[:, None]
    dinvb[...] = dinv
    y0[...] = jnp.dot(xb[...], w0[...], preferred_element_type=jnp.float32) * dinv
    lin0[...] = jnp.maximum(
        jnp.dot(f0b[...], wl0[...], preferred_element_type=jnp.float32) + bl0[...], 0.0)


def _tc1_body(p0, lin0, f1b, f2b, dinvb, w1, wl1a, wl1b, b0, bl1, y1, lin1):
    dinv = dinvb[...]
    conv0 = jnp.maximum(dinv * (p0[0] + p0[1]) + b0[...], 0.0)
    comb = jnp.concatenate([lin0[...], conv0], axis=1)
    y1[...] = jnp.dot(comb, w1[...], preferred_element_type=jnp.float32) * dinv
    lin1[...] = jnp.maximum(
        jnp.dot(f1b[...], wl1a[...], preferred_element_type=jnp.float32)
        + jnp.dot(f2b[...], wl1b[...], preferred_element_type=jnp.float32)
        + bl1[...], 0.0)


def _tc2_body(p1, lin1, dinvb, wc, b1, yc):
    dinv = dinvb[...]
    conv1 = jnp.maximum(dinv * (p1[0] + p1[1]) + b1[...], 0.0)
    comb = jnp.concatenate([lin1[...], conv1], axis=1)
    yc[...] = jnp.dot(comb, wc[...], preferred_element_type=jnp.float32) * dinv


def _tc3_body(pc, dinvb, bc, out0, convc):
    v = dinvb[...] * (pc[0] + pc[1]) + bc[...]
    m = jnp.max(v, axis=-1, keepdims=True)
    lse = jnp.log(jnp.sum(jnp.exp(v - m), axis=-1, keepdims=True)) + m
    convc[...] = v
    out0[...] = v - lse


def _row_spec(d):
    return pl.BlockSpec((_BLK, d), lambda i: (i, 0))


def _pair_spec(d):
    return pl.BlockSpec((2, _BLK, d), lambda i: (0, i, 0))


def _full_spec(shape):
    return pl.BlockSpec(shape, lambda i: tuple(0 for _ in shape))


def kernel(x, edge_index, features, W0, b0, Wl0, bl0, W1, b1, Wl1, bl1,
           order_weights, Wc, bc):
    n, fin = x.shape
    nhid = W0.shape[1]
    ncls = Wc.shape[1]
    nc, ns = _sc_info()
    nw = nc * ns

    npad = ((n + 1 + _BLK - 1) // _BLK) * _BLK
    grid = npad // _BLK

    idt = edge_index.dtype
    loop = jnp.arange(n, dtype=idt)
    src = jnp.concatenate([edge_index[0], loop])
    dst = jnp.concatenate([edge_index[1], loop])
    et = src.shape[0]
    nch = -(-et // (nw * _CH))
    nch = -(-nch // _NBUF) * _NBUF
    epad = nw * nch * _CH
    srcp = jnp.full((epad,), n, idt).at[:et].set(src).reshape(nw, nch, _CH)
    dstp = jnp.full((epad,), n, idt).at[:et].set(dst).reshape(nw, nch, _CH)

    xp = jnp.zeros((npad, fin), jnp.float32).at[:n].set(x)
    f0p = jnp.zeros((npad, fin), jnp.float32).at[:n].set(features[0])
    f1p = jnp.zeros((npad, fin), jnp.float32).at[:n].set(features[1])
    f2p = jnp.zeros((npad, fin), jnp.float32).at[:n].set(features[2])

    wl1a = order_weights[0] * Wl1[:fin]
    wl1b = order_weights[1] * Wl1[fin:]
    b0r = b0[None, :]
    b1r = b1[None, :]
    bl0r = bl0[None, :]
    bl1r = bl1[None, :]
    bcr = bc[None, :]

    zdeg = jnp.zeros((npad,), jnp.float32)
    zh = jnp.zeros((npad, nhid), jnp.float32)
    zc = jnp.zeros((npad, ncls), jnp.float32)

    deg_fn = _make_deg(npad, nch, nc, ns)
    spmm_h = _make_spmm(npad, nhid, nch, nc, ns)
    spmm_c = _make_spmm(npad, ncls, nch, nc, ns)

    degp = deg_fn(dstp, zdeg)

    y0, lin0, dinv = pl.pallas_call(
        _tc0_body,
        grid=(grid,),
        in_specs=[
            pl.BlockSpec((2, _BLK), lambda i: (0, i)),
            _row_spec(fin), _row_spec(fin),
            _full_spec((fin, nhid)), _full_spec((fin, nhid)),
            _full_spec((1, nhid)),
        ],
        out_specs=[_row_spec(nhid), _row_spec(nhid), _row_spec(1)],
        out_shape=[
            jax.ShapeDtypeStruct((npad, nhid), jnp.float32),
            jax.ShapeDtypeStruct((npad, nhid), jnp.float32),
            jax.ShapeDtypeStruct((npad, 1), jnp.float32),
        ],
    )(degp, xp, f0p, W0, Wl0, bl0r)

    p0 = spmm_h(y0, srcp, dstp, zh)

    y1, lin1 = pl.pallas_call(
        _tc1_body,
        grid=(grid,),
        in_specs=[
            _pair_spec(nhid), _row_spec(nhid), _row_spec(fin), _row_spec(fin),
            _row_spec(1),
            _full_spec((fin, nhid)), _full_spec((fin, nhid)),
            _full_spec((fin, nhid)),
            _full_spec((1, nhid)), _full_spec((1, nhid)),
        ],
        out_specs=[_row_spec(nhid), _row_spec(nhid)],
        out_shape=[
            jax.ShapeDtypeStruct((npad, nhid), jnp.float32),
            jax.ShapeDtypeStruct((npad, nhid), jnp.float32),
        ],
    )(p0, lin0, f1p, f2p, dinv, W1, wl1a, wl1b, b0r, bl1r)

    p1 = spmm_h(y1, srcp, dstp, zh)

    yc = pl.pallas_call(
        _tc2_body,
        grid=(grid,),
        in_specs=[
            _pair_spec(nhid), _row_spec(nhid), _row_spec(1),
            _full_spec((2 * nhid, ncls)), _full_spec((1, nhid)),
        ],
        out_specs=_row_spec(ncls),
        out_shape=jax.ShapeDtypeStruct((npad, ncls), jnp.float32),
    )(p1, lin1, dinv, Wc, b1r)

    pc = spmm_c(yc, srcp, dstp, zc)

    out0, convc = pl.pallas_call(
        _tc3_body,
        grid=(grid,),
        in_specs=[_pair_spec(ncls), _row_spec(1), _full_spec((1, ncls))],
        out_specs=[_row_spec(ncls), _row_spec(ncls)],
        out_shape=[
            jax.ShapeDtypeStruct((npad, ncls), jnp.float32),
            jax.ShapeDtypeStruct((npad, ncls), jnp.float32),
        ],
    )(pc, dinv, bcr)

    return (out0[:n], convc[:n])


# speedup vs baseline: 2.7229x; 1.8541x over previous
import functools

import jax
import jax.numpy as jnp
from jax import lax
from jax.experimental import pallas as pl
from jax.experimental.pallas import tpu as pltpu
from jax.experimental.pallas import tpu_sc as plsc

_CH = 128
_NBUF = 2
_BLK = 1024


def _sc_info():
    try:
        info = plsc.get_sparse_core_info()
        return info.num_cores, info.num_subcores
    except Exception:
        return 2, 16


@functools.lru_cache(maxsize=None)
def _make_deg(npad, nch, nc, ns):
    mesh = plsc.VectorSubcoreMesh(core_axis_name="c", subcore_axis_name="s",
                                  num_cores=nc, num_subcores=ns)
    rows_per_tile = npad // ns

    def body(dst_hbm, zero_hbm, out_hbm, dstv, ones_v, acc):
        c = lax.axis_index("c")
        s = lax.axis_index("s")
        wid = s * nc + c
        pltpu.sync_copy(dst_hbm.at[wid], dstv)
        for i in range(_CH // 16):
            ones_v[pl.ds(i * 16, 16)] = jnp.full((16,), 1.0, jnp.float32)

        @pl.when(s == 0)
        def _():
            pltpu.sync_copy(zero_hbm, acc)

        plsc.subcore_barrier()

        def step(a, carry):
            pltpu.sync_copy(ones_v, acc.at[dstv.at[a]], add=True)
            return carry

        lax.fori_loop(0, nch, step, 0)
        plsc.subcore_barrier()
        lo = s * rows_per_tile
        pltpu.sync_copy(acc.at[pl.ds(lo, rows_per_tile)],
                        out_hbm.at[c].at[pl.ds(lo, rows_per_tile)])

    return pl.kernel(
        body,
        out_type=jax.ShapeDtypeStruct((nc, npad), jnp.float32),
        mesh=mesh,
        compiler_params=pltpu.CompilerParams(use_tc_tiling_on_sc=False),
        scratch_types=[
            pltpu.VMEM((nch, _CH), jnp.int32),
            pltpu.VMEM((_CH,), jnp.float32),
            pltpu.VMEM_SHARED((npad,), jnp.float32),
        ],
    )


@functools.lru_cache(maxsize=None)
def _make_spmm(npad, d, nch, nc, ns):
    mesh = plsc.VectorSubcoreMesh(core_axis_name="c", subcore_axis_name="s",
                                  num_cores=nc, num_subcores=ns)
    rows_per_tile = npad // ns

    nbuf = _NBUF

    def body(y_hbm, src_hbm, dst_hbm, zero_hbm, out_hbm,
             srcv, dstv, rows, gsems, acc, yspm):
        c = lax.axis_index("c")
        s = lax.axis_index("s")
        wid = s * nc + c
        lo = s * rows_per_tile
        pltpu.sync_copy(src_hbm.at[wid], srcv)
        pltpu.sync_copy(dst_hbm.at[wid], dstv)

        pltpu.sync_copy(y_hbm.at[pl.ds(lo, rows_per_tile)],
                        yspm.at[pl.ds(lo, rows_per_tile)])

        @pl.when(s == 0)
        def _():
            pltpu.sync_copy(zero_hbm, acc)

        plsc.subcore_barrier()

        for b in range(nbuf - 1):
            pltpu.async_copy(yspm.at[srcv.at[b]], rows[b], gsems[b])

        def outer(g2, carry):
            g = g2 * nbuf
            for b in range(nbuf):
                a = g + b
                pltpu.make_async_copy(yspm.at[srcv.at[a]], rows[b],
                                      gsems[b]).wait()

                nb = (b + nbuf - 1) % nbuf

                @pl.when(a + nbuf - 1 < nch)
                def _():
                    pltpu.async_copy(yspm.at[srcv.at[a + nbuf - 1]],
                                     rows[nb], gsems[nb])

                pltpu.sync_copy(rows[b], acc.at[dstv.at[a]], add=True)
            return carry

        lax.fori_loop(0, nch // nbuf, outer, 0)
        plsc.subcore_barrier()
        pltpu.sync_copy(acc.at[pl.ds(lo, rows_per_tile)],
                        out_hbm.at[c].at[pl.ds(lo, rows_per_tile)])

    return pl.kernel(
        body,
        out_type=jax.ShapeDtypeStruct((nc, npad, d), jnp.float32),
        mesh=mesh,
        compiler_params=pltpu.CompilerParams(use_tc_tiling_on_sc=False),
        scratch_types=[
            pltpu.VMEM((nch, _CH), jnp.int32),
            pltpu.VMEM((nch, _CH), jnp.int32),
            [pltpu.VMEM((_CH, d), jnp.float32) for _ in range(nbuf)],
            [pltpu.SemaphoreType.DMA for _ in range(nbuf)],
            pltpu.VMEM_SHARED((npad, d), jnp.float32),
            pltpu.VMEM_SHARED((npad, d), jnp.float32),
        ],
    )


def _tc0_body(degp, xb, f0b, w0, wl0, bl0, y0, lin0, dinvb):
    deg = degp[0, :] + degp[1, :]
    dinv = lax.rsqrt(jnp.maximum(deg, 1.0))[:, None]
    dinvb[...] = dinv
    y0[...] = jnp.dot(xb[...], w0[...], preferred_element_type=jnp.float32) * dinv
    lin0[...] = jnp.maximum(
        jnp.dot(f0b[...], wl0[...], preferred_element_type=jnp.float32) + bl0[...], 0.0)


def _tc1_body(p0, lin0, f1b, f2b, dinvb, w1, wl1a, wl1b, b0, bl1, y1, lin1):
    dinv = dinvb[...]
    conv0 = jnp.maximum(dinv * (p0[0] + p0[1]) + b0[...], 0.0)
    comb = jnp.concatenate([lin0[...], conv0], axis=1)
    y1[...] = jnp.dot(comb, w1[...], preferred_element_type=jnp.float32) * dinv
    lin1[...] = jnp.maximum(
        jnp.dot(f1b[...], wl1a[...], preferred_element_type=jnp.float32)
        + jnp.dot(f2b[...], wl1b[...], preferred_element_type=jnp.float32)
        + bl1[...], 0.0)


def _tc2_body(p1, lin1, dinvb, wc, b1, yc):
    dinv = dinvb[...]
    conv1 = jnp.maximum(dinv * (p1[0] + p1[1]) + b1[...], 0.0)
    comb = jnp.concatenate([lin1[...], conv1], axis=1)
    yc[...] = jnp.dot(comb, wc[...], preferred_element_type=jnp.float32) * dinv


def _tc3_body(pc, dinvb, bc, out0, convc):
    v = dinvb[...] * (pc[0] + pc[1]) + bc[...]
    m = jnp.max(v, axis=-1, keepdims=True)
    lse = jnp.log(jnp.sum(jnp.exp(v - m), axis=-1, keepdims=True)) + m
    convc[...] = v
    out0[...] = v - lse


def _row_spec(d):
    return pl.BlockSpec((_BLK, d), lambda i: (i, 0))


def _pair_spec(d):
    return pl.BlockSpec((2, _BLK, d), lambda i: (0, i, 0))


def _full_spec(shape):
    return pl.BlockSpec(shape, lambda i: tuple(0 for _ in shape))


def kernel(x, edge_index, features, W0, b0, Wl0, bl0, W1, b1, Wl1, bl1,
           order_weights, Wc, bc):
    n, fin = x.shape
    nhid = W0.shape[1]
    ncls = Wc.shape[1]
    nc, ns = _sc_info()
    nw = nc * ns

    npad = ((n + 1 + _BLK - 1) // _BLK) * _BLK
    grid = npad // _BLK

    idt = edge_index.dtype
    loop = jnp.arange(n, dtype=idt)
    src = jnp.concatenate([edge_index[0], loop])
    dst = jnp.concatenate([edge_index[1], loop])
    et = src.shape[0]
    nch = -(-et // (nw * _CH))
    nch = -(-nch // _NBUF) * _NBUF
    epad = nw * nch * _CH
    srcp = jnp.full((epad,), n, idt).at[:et].set(src).reshape(nw, nch, _CH)
    dstp = jnp.full((epad,), n, idt).at[:et].set(dst).reshape(nw, nch, _CH)

    xp = jnp.zeros((npad, fin), jnp.float32).at[:n].set(x)
    f0p = jnp.zeros((npad, fin), jnp.float32).at[:n].set(features[0])
    f1p = jnp.zeros((npad, fin), jnp.float32).at[:n].set(features[1])
    f2p = jnp.zeros((npad, fin), jnp.float32).at[:n].set(features[2])

    wl1a = order_weights[0] * Wl1[:fin]
    wl1b = order_weights[1] * Wl1[fin:]
    b0r = b0[None, :]
    b1r = b1[None, :]
    bl0r = bl0[None, :]
    bl1r = bl1[None, :]
    bcr = bc[None, :]

    zdeg = jnp.zeros((npad,), jnp.float32)
    zh = jnp.zeros((npad, nhid), jnp.float32)
    zc = jnp.zeros((npad, ncls), jnp.float32)

    deg_fn = _make_deg(npad, nch, nc, ns)
    spmm_h = _make_spmm(npad, nhid, nch, nc, ns)
    spmm_c = _make_spmm(npad, ncls, nch, nc, ns)

    degp = deg_fn(dstp, zdeg)

    y0, lin0, dinv = pl.pallas_call(
        _tc0_body,
        grid=(grid,),
        in_specs=[
            pl.BlockSpec((2, _BLK), lambda i: (0, i)),
            _row_spec(fin), _row_spec(fin),
            _full_spec((fin, nhid)), _full_spec((fin, nhid)),
            _full_spec((1, nhid)),
        ],
        out_specs=[_row_spec(nhid), _row_spec(nhid), _row_spec(1)],
        out_shape=[
            jax.ShapeDtypeStruct((npad, nhid), jnp.float32),
            jax.ShapeDtypeStruct((npad, nhid), jnp.float32),
            jax.ShapeDtypeStruct((npad, 1), jnp.float32),
        ],
    )(degp, xp, f0p, W0, Wl0, bl0r)

    p0 = spmm_h(y0, srcp, dstp, zh)

    y1, lin1 = pl.pallas_call(
        _tc1_body,
        grid=(grid,),
        in_specs=[
            _pair_spec(nhid), _row_spec(nhid), _row_spec(fin), _row_spec(fin),
            _row_spec(1),
            _full_spec((fin, nhid)), _full_spec((fin, nhid)),
            _full_spec((fin, nhid)),
            _full_spec((1, nhid)), _full_spec((1, nhid)),
        ],
        out_specs=[_row_spec(nhid), _row_spec(nhid)],
        out_shape=[
            jax.ShapeDtypeStruct((npad, nhid), jnp.float32),
            jax.ShapeDtypeStruct((npad, nhid), jnp.float32),
        ],
    )(p0, lin0, f1p, f2p, dinv, W1, wl1a, wl1b, b0r, bl1r)

    p1 = spmm_h(y1, srcp, dstp, zh)

    yc = pl.pallas_call(
        _tc2_body,
        grid=(grid,),
        in_specs=[
            _pair_spec(nhid), _row_spec(nhid), _row_spec(1),
            _full_spec((2 * nhid, ncls)), _full_spec((1, nhid)),
        ],
        out_specs=_row_spec(ncls),
        out_shape=jax.ShapeDtypeStruct((npad, ncls), jnp.float32),
    )(p1, lin1, dinv, Wc, b1r)

    pc = spmm_c(yc, srcp, dstp, zc)

    out0, convc = pl.pallas_call(
        _tc3_body,
        grid=(grid,),
        in_specs=[_pair_spec(ncls), _row_spec(1), _full_spec((1, ncls))],
        out_specs=[_row_spec(ncls), _row_spec(ncls)],
        out_shape=[
            jax.ShapeDtypeStruct((npad, ncls), jnp.float32),
            jax.ShapeDtypeStruct((npad, ncls), jnp.float32),
        ],
    )(pc, dinv, bcr)

    return (out0[:n], convc[:n])


# speedup vs baseline: 3.0192x; 1.1088x over previous
import functools

import jax
import jax.numpy as jnp
from jax import lax
from jax.experimental import pallas as pl
from jax.experimental.pallas import tpu as pltpu
from jax.experimental.pallas import tpu_sc as plsc

_CH = 128
_NBUF = 2
_BLK = 2048


def _sc_info():
    try:
        info = plsc.get_sparse_core_info()
        return info.num_cores, info.num_subcores
    except Exception:
        return 2, 16


@functools.lru_cache(maxsize=None)
def _make_deg(npad, nch, nc, ns):
    mesh = plsc.VectorSubcoreMesh(core_axis_name="c", subcore_axis_name="s",
                                  num_cores=nc, num_subcores=ns)
    rows_per_tile = npad // ns

    def body(dst_hbm, zero_hbm, out_hbm, dstv, ones_v, acc):
        c = lax.axis_index("c")
        s = lax.axis_index("s")
        wid = s * nc + c
        pltpu.sync_copy(dst_hbm.at[wid], dstv)
        for i in range(_CH // 16):
            ones_v[pl.ds(i * 16, 16)] = jnp.full((16,), 1.0, jnp.float32)

        @pl.when(s == 0)
        def _():
            pltpu.sync_copy(zero_hbm, acc)

        plsc.subcore_barrier()

        def step(a, carry):
            pltpu.sync_copy(ones_v, acc.at[dstv.at[a]], add=True)
            return carry

        lax.fori_loop(0, nch, step, 0)
        plsc.subcore_barrier()
        lo = s * rows_per_tile
        pltpu.sync_copy(acc.at[pl.ds(lo, rows_per_tile)],
                        out_hbm.at[c].at[pl.ds(lo, rows_per_tile)])

    return pl.kernel(
        body,
        out_type=jax.ShapeDtypeStruct((nc, npad), jnp.float32),
        mesh=mesh,
        compiler_params=pltpu.CompilerParams(use_tc_tiling_on_sc=False),
        scratch_types=[
            pltpu.VMEM((nch, _CH), jnp.int32),
            pltpu.VMEM((_CH,), jnp.float32),
            pltpu.VMEM_SHARED((npad,), jnp.float32),
        ],
    )


@functools.lru_cache(maxsize=None)
def _make_spmm(npad, d, nch, nc, ns):
    mesh = plsc.VectorSubcoreMesh(core_axis_name="c", subcore_axis_name="s",
                                  num_cores=nc, num_subcores=ns)
    rows_per_tile = npad // ns
    nbuf = _NBUF

    def body(y_hbm, src_hbm, dst_hbm, out_hbm,
             srcv, dstv, rows, gsems, acc, yspm):
        c = lax.axis_index("c")
        s = lax.axis_index("s")
        wid = s * nc + c
        lo = s * rows_per_tile
        pltpu.sync_copy(src_hbm.at[wid], srcv)
        pltpu.sync_copy(dst_hbm.at[wid], dstv)

        zvec = jnp.zeros((16,), jnp.float32)

        def zrow(i, carry):
            for cc in range(d // 16):
                rows[0][i, pl.ds(cc * 16, 16)] = zvec
            return carry

        lax.fori_loop(0, _CH, zrow, 0)
        for k in range(rows_per_tile // _CH):
            pltpu.sync_copy(rows[0], acc.at[pl.ds(lo + k * _CH, _CH)])

        pltpu.sync_copy(y_hbm.at[pl.ds(lo, rows_per_tile), pl.ds(0, d)],
                        yspm.at[pl.ds(lo, rows_per_tile)])

        plsc.subcore_barrier()

        for b in range(nbuf - 1):
            pltpu.async_copy(yspm.at[srcv.at[b]], rows[b], gsems[b])

        def outer(g2, carry):
            g = g2 * nbuf
            for b in range(nbuf):
                a = g + b
                pltpu.make_async_copy(yspm.at[srcv.at[a]], rows[b],
                                      gsems[b]).wait()

                nb = (b + nbuf - 1) % nbuf

                @pl.when(a + nbuf - 1 < nch)
                def _():
                    pltpu.async_copy(yspm.at[srcv.at[a + nbuf - 1]],
                                     rows[nb], gsems[nb])

                pltpu.sync_copy(rows[b], acc.at[dstv.at[a]], add=True)
            return carry

        lax.fori_loop(0, nch // nbuf, outer, 0)
        plsc.subcore_barrier()
        pltpu.sync_copy(acc.at[pl.ds(lo, rows_per_tile)],
                        out_hbm.at[pl.ds(lo, rows_per_tile),
                                   pl.ds(c * 64, d)])

    return pl.kernel(
        body,
        out_type=jax.ShapeDtypeStruct((npad, 128), jnp.float32),
        mesh=mesh,
        compiler_params=pltpu.CompilerParams(use_tc_tiling_on_sc=False),
        scratch_types=[
            pltpu.VMEM((nch, _CH), jnp.int32),
            pltpu.VMEM((nch, _CH), jnp.int32),
            [pltpu.VMEM((_CH, d), jnp.float32) for _ in range(nbuf)],
            [pltpu.SemaphoreType.DMA for _ in range(nbuf)],
            pltpu.VMEM_SHARED((npad, d), jnp.float32),
            pltpu.VMEM_SHARED((npad, d), jnp.float32),
        ],
    )


def _tc_pre_body(xb, f0b, f1b, f2b, w0, wl0, bl0, wl1a, wl1b, bl1, w1a, wca,
                 xw0, pre1, pre2):
    lin0 = jnp.maximum(
        jnp.dot(f0b[...], wl0[...], preferred_element_type=jnp.float32)
        + bl0[...], 0.0)
    lin1 = jnp.maximum(
        jnp.dot(f1b[...], wl1a[...], preferred_element_type=jnp.float32)
        + jnp.dot(f2b[...], wl1b[...], preferred_element_type=jnp.float32)
        + bl1[...], 0.0)
    xw0[...] = jnp.dot(xb[...], w0[...], preferred_element_type=jnp.float32)
    pre1[...] = jnp.dot(lin0, w1a[...], preferred_element_type=jnp.float32)
    pre2[...] = jnp.dot(lin1, wca[...], preferred_element_type=jnp.float32)


def _pad128(v):
    return jnp.concatenate(
        [v, jnp.zeros((v.shape[0], 128 - v.shape[1]), jnp.float32)], axis=1)


def _tc_scale_body(degp, xw0, y0, dinvb):
    deg = degp[0, :] + degp[1, :]
    dinv = lax.rsqrt(jnp.maximum(deg, 1.0))[:, None]
    dinvb[...] = dinv
    y0[...] = _pad128(xw0[...] * dinv)


def _tc_mid1_body(p0, pre1, dinvb, b0, w1b, y1):
    dinv = dinvb[...]
    p = p0[...]
    conv0 = jnp.maximum(dinv * (p[:, :64] + p[:, 64:]) + b0[...], 0.0)
    y1[...] = _pad128(
        (pre1[...] + jnp.dot(conv0, w1b[...],
                             preferred_element_type=jnp.float32)) * dinv)


def _tc_mid2_body(p1, pre2, dinvb, b1, wcb, yc):
    dinv = dinvb[...]
    p = p1[...]
    conv1 = jnp.maximum(dinv * (p[:, :64] + p[:, 64:]) + b1[...], 0.0)
    v = (pre2[...] + jnp.dot(conv1, wcb[...],
                             preferred_element_type=jnp.float32)) * dinv
    yc[...] = _pad128(v)


def _tc_post_body(pc, dinvb, bc, out0, convc):
    ncls = bc.shape[1]
    p = pc[...]
    v = dinvb[...] * (p[:, :ncls] + p[:, 64:64 + ncls]) + bc[...]
    m = jnp.max(v, axis=-1, keepdims=True)
    lse = jnp.log(jnp.sum(jnp.exp(v - m), axis=-1, keepdims=True)) + m
    convc[...] = v
    out0[...] = v - lse


def _row_spec(d, blk=_BLK):
    return pl.BlockSpec((blk, d), lambda i: (i, 0))


def _full_spec(shape):
    return pl.BlockSpec(shape, lambda i: tuple(0 for _ in shape))


def kernel(x, edge_index, features, W0, b0, Wl0, bl0, W1, b1, Wl1, bl1,
           order_weights, Wc, bc):
    n, fin = x.shape
    nhid = W0.shape[1]
    ncls = Wc.shape[1]
    ncpad = 48
    nc, ns = _sc_info()
    nw = nc * ns

    npad = ((n + 1 + _BLK - 1) // _BLK) * _BLK
    grid = npad // _BLK

    idt = edge_index.dtype
    loop = jnp.arange(n, dtype=idt)
    src = jnp.concatenate([edge_index[0], loop])
    dst = jnp.concatenate([edge_index[1], loop])
    et = src.shape[0]
    nch = -(-et // (nw * _CH))
    nch = -(-nch // _NBUF) * _NBUF
    epad = nw * nch * _CH
    srcp = jnp.full((epad,), n, idt).at[:et].set(src).reshape(nw, nch, _CH)
    dstp = jnp.full((epad,), n, idt).at[:et].set(dst).reshape(nw, nch, _CH)

    xp = jnp.zeros((npad, fin), jnp.float32).at[:n].set(x)
    f0p = jnp.zeros((npad, fin), jnp.float32).at[:n].set(features[0])
    f1p = jnp.zeros((npad, fin), jnp.float32).at[:n].set(features[1])
    f2p = jnp.zeros((npad, fin), jnp.float32).at[:n].set(features[2])

    wl1a = order_weights[0] * Wl1[:fin]
    wl1b = order_weights[1] * Wl1[fin:]
    w1a, w1b = W1[:nhid], W1[nhid:]
    wca, wcb = Wc[:nhid], Wc[nhid:]
    b0r = b0[None, :]
    b1r = b1[None, :]
    bl0r = bl0[None, :]
    bl1r = bl1[None, :]
    bcr = bc[None, :]

    zdeg = jnp.zeros((npad,), jnp.float32)

    deg_fn = _make_deg(npad, nch, nc, ns)
    spmm_h = _make_spmm(npad, nhid, nch, nc, ns)
    spmm_c = _make_spmm(npad, ncpad, nch, nc, ns)

    degp = deg_fn(dstp, zdeg)

    xw0, pre1, pre2 = pl.pallas_call(
        _tc_pre_body,
        grid=(grid,),
        in_specs=[
            _row_spec(fin), _row_spec(fin), _row_spec(fin), _row_spec(fin),
            _full_spec((fin, nhid)), _full_spec((fin, nhid)),
            _full_spec((1, nhid)),
            _full_spec((fin, nhid)), _full_spec((fin, nhid)),
            _full_spec((1, nhid)),
            _full_spec((nhid, nhid)), _full_spec((nhid, ncls)),
        ],
        out_specs=[_row_spec(nhid), _row_spec(nhid), _row_spec(ncls)],
        out_shape=[
            jax.ShapeDtypeStruct((npad, nhid), jnp.float32),
            jax.ShapeDtypeStruct((npad, nhid), jnp.float32),
            jax.ShapeDtypeStruct((npad, ncls), jnp.float32),
        ],
    )(xp, f0p, f1p, f2p, W0, Wl0, bl0r, wl1a, wl1b, bl1r, w1a, wca)

    y0, dinv = pl.pallas_call(
        _tc_scale_body,
        grid=(grid,),
        in_specs=[pl.BlockSpec((2, _BLK), lambda i: (0, i)), _row_spec(nhid)],
        out_specs=[_row_spec(128), _row_spec(1)],
        out_shape=[
            jax.ShapeDtypeStruct((npad, 128), jnp.float32),
            jax.ShapeDtypeStruct((npad, 1), jnp.float32),
        ],
    )(degp, xw0)

    p0 = spmm_h(y0, srcp, dstp)

    y1 = pl.pallas_call(
        _tc_mid1_body,
        grid=(grid,),
        in_specs=[
            _row_spec(128), _row_spec(nhid), _row_spec(1),
            _full_spec((1, nhid)), _full_spec((nhid, nhid)),
        ],
        out_specs=_row_spec(128),
        out_shape=jax.ShapeDtypeStruct((npad, 128), jnp.float32),
    )(p0, pre1, dinv, b0r, w1b)

    p1 = spmm_h(y1, srcp, dstp)

    yc = pl.pallas_call(
        _tc_mid2_body,
        grid=(grid,),
        in_specs=[
            _row_spec(128), _row_spec(ncls), _row_spec(1),
            _full_spec((1, nhid)), _full_spec((nhid, ncls)),
        ],
        out_specs=_row_spec(128),
        out_shape=jax.ShapeDtypeStruct((npad, 128), jnp.float32),
    )(p1, pre2, dinv, b1r, wcb)

    pc = spmm_c(yc, srcp, dstp)

    blkp = 2000
    out0, convc = pl.pallas_call(
        _tc_post_body,
        grid=(n // blkp,),
        in_specs=[
            _row_spec(128, blkp), _row_spec(1, blkp), _full_spec((1, ncls)),
        ],
        out_specs=[_row_spec(ncls, blkp), _row_spec(ncls, blkp)],
        out_shape=[
            jax.ShapeDtypeStruct((n, ncls), jnp.float32),
            jax.ShapeDtypeStruct((n, ncls), jnp.float32),
        ],
    )(pc, dinv, bcr)

    return (out0, convc)


# speedup vs baseline: 3.1959x; 1.0585x over previous
import functools

import jax
import jax.numpy as jnp
from jax import lax
from jax.experimental import pallas as pl
from jax.experimental.pallas import tpu as pltpu
from jax.experimental.pallas import tpu_sc as plsc

_CH = 128
_NBUF = 2
_BLK = 2048


def _sc_info():
    try:
        info = plsc.get_sparse_core_info()
        return info.num_cores, info.num_subcores
    except Exception:
        return 2, 16


@functools.lru_cache(maxsize=None)
def _make_deg(npad, nch, nc, ns):
    mesh = plsc.VectorSubcoreMesh(core_axis_name="c", subcore_axis_name="s",
                                  num_cores=nc, num_subcores=ns)
    rows_per_tile = npad // ns

    def body(dst_hbm, zero_hbm, out_hbm, dstv, ones_v, acc):
        c = lax.axis_index("c")
        s = lax.axis_index("s")
        wid = s * nc + c
        pltpu.sync_copy(dst_hbm.at[wid], dstv)
        for i in range(_CH // 16):
            ones_v[pl.ds(i * 16, 16)] = jnp.full((16,), 1.0, jnp.float32)

        @pl.when(s == 0)
        def _():
            pltpu.sync_copy(zero_hbm, acc)

        plsc.subcore_barrier()

        def step(a, carry):
            pltpu.sync_copy(ones_v, acc.at[dstv.at[a]], add=True)
            return carry

        lax.fori_loop(0, nch, step, 0)
        plsc.subcore_barrier()
        lo = s * rows_per_tile
        pltpu.sync_copy(acc.at[pl.ds(lo, rows_per_tile)],
                        out_hbm.at[c].at[pl.ds(lo, rows_per_tile)])

    return pl.kernel(
        body,
        out_type=jax.ShapeDtypeStruct((nc, npad), jnp.float32),
        mesh=mesh,
        compiler_params=pltpu.CompilerParams(use_tc_tiling_on_sc=False),
        scratch_types=[
            pltpu.VMEM((nch, _CH), jnp.int32),
            pltpu.VMEM((_CH,), jnp.float32),
            pltpu.VMEM_SHARED((npad,), jnp.float32),
        ],
    )


@functools.lru_cache(maxsize=None)
def _make_spmm(npad, d, nch, nc, ns):
    mesh = plsc.VectorSubcoreMesh(core_axis_name="c", subcore_axis_name="s",
                                  num_cores=nc, num_subcores=ns)
    rows_per_tile = npad // ns
    nbuf = _NBUF

    def body(y_hbm, src_hbm, dst_hbm, out_hbm,
             srcv, dstv, rows, gsems, acc, yspm):
        c = lax.axis_index("c")
        s = lax.axis_index("s")
        wid = s * nc + c
        lo = s * rows_per_tile
        pltpu.sync_copy(src_hbm.at[wid], srcv)
        pltpu.sync_copy(dst_hbm.at[wid], dstv)

        zvec = jnp.zeros((16,), jnp.float32)

        def zrow(i, carry):
            for cc in range(d // 16):
                rows[0][i, pl.ds(cc * 16, 16)] = zvec
            return carry

        lax.fori_loop(0, _CH, zrow, 0)
        for k in range(rows_per_tile // _CH):
            pltpu.sync_copy(rows[0], acc.at[pl.ds(lo + k * _CH, _CH)])

        pltpu.sync_copy(y_hbm.at[pl.ds(lo, rows_per_tile), pl.ds(0, d)],
                        yspm.at[pl.ds(lo, rows_per_tile)])

        plsc.subcore_barrier()

        for b in range(nbuf - 1):
            pltpu.async_copy(yspm.at[srcv.at[b]], rows[b], gsems[b])

        def outer(g2, carry):
            g = g2 * nbuf
            for b in range(nbuf):
                a = g + b
                pltpu.make_async_copy(yspm.at[srcv.at[a]], rows[b],
                                      gsems[b]).wait()

                nb = (b + nbuf - 1) % nbuf

                @pl.when(a + nbuf - 1 < nch)
                def _():
                    pltpu.async_copy(yspm.at[srcv.at[a + nbuf - 1]],
                                     rows[nb], gsems[nb])

                pltpu.sync_copy(rows[b], acc.at[dstv.at[a]], add=True)
            return carry

        lax.fori_loop(0, nch // nbuf, outer, 0)
        plsc.subcore_barrier()
        pltpu.sync_copy(acc.at[pl.ds(lo, rows_per_tile)],
                        out_hbm.at[pl.ds(lo, rows_per_tile),
                                   pl.ds(c * 64, d)])

    return pl.kernel(
        body,
        out_type=jax.ShapeDtypeStruct((npad, 128), jnp.float32),
        mesh=mesh,
        compiler_params=pltpu.CompilerParams(use_tc_tiling_on_sc=False),
        scratch_types=[
            pltpu.VMEM((nch, _CH), jnp.int32),
            pltpu.VMEM((nch, _CH), jnp.int32),
            [pltpu.VMEM((_CH, d), jnp.float32) for _ in range(nbuf)],
            [pltpu.SemaphoreType.DMA for _ in range(nbuf)],
            pltpu.VMEM_SHARED((npad, d), jnp.float32),
            pltpu.VMEM_SHARED((npad, d), jnp.float32),
        ],
    )


def _pad128(v):
    return jnp.concatenate(
        [v, jnp.zeros((v.shape[0], 128 - v.shape[1]), jnp.float32)], axis=1)


def _tc_pre_body(degp, xb, fb, w0, wl0, bl0, wl1a, wl1b, bl1, w1a, wca,
                 y0, dinvb, pre1, pre2):
    lin0 = jnp.maximum(
        jnp.dot(fb[0], wl0[...], preferred_element_type=jnp.float32)
        + bl0[...], 0.0)
    lin1 = jnp.maximum(
        jnp.dot(fb[1], wl1a[...], preferred_element_type=jnp.float32)
        + jnp.dot(fb[2], wl1b[...], preferred_element_type=jnp.float32)
        + bl1[...], 0.0)
    xw0 = jnp.dot(xb[...], w0[...], preferred_element_type=jnp.float32)
    pre1[...] = jnp.dot(lin0, w1a[...], preferred_element_type=jnp.float32)
    pre2[...] = jnp.dot(lin1, wca[...], preferred_element_type=jnp.float32)
    deg = degp[0, :] + degp[1, :]
    dinv = lax.rsqrt(jnp.maximum(deg, 1.0))[:, None]
    dinvb[...] = dinv
    y0[...] = _pad128(xw0 * dinv)


def _tc_mid1_body(p0, pre1, dinvb, b0, w1b, y1):
    dinv = dinvb[...]
    p = p0[...]
    conv0 = jnp.maximum(dinv * (p[:, :64] + p[:, 64:]) + b0[...], 0.0)
    y1[...] = _pad128(
        (pre1[...] + jnp.dot(conv0, w1b[...],
                             preferred_element_type=jnp.float32)) * dinv)


def _tc_mid2_body(p1, pre2, dinvb, b1, wcb, yc):
    dinv = dinvb[...]
    p = p1[...]
    conv1 = jnp.maximum(dinv * (p[:, :64] + p[:, 64:]) + b1[...], 0.0)
    v = (pre2[...] + jnp.dot(conv1, wcb[...],
                             preferred_element_type=jnp.float32)) * dinv
    yc[...] = _pad128(v)


def _tc_post_body(pc, dinvb, bc, out0, convc):
    ncls = bc.shape[1]
    p = pc[...]
    v = dinvb[...] * (p[:, :ncls] + p[:, 64:64 + ncls]) + bc[...]
    m = jnp.max(v, axis=-1, keepdims=True)
    lse = jnp.log(jnp.sum(jnp.exp(v - m), axis=-1, keepdims=True)) + m
    convc[...] = v
    out0[...] = v - lse


def _row_spec(d, blk=_BLK):
    return pl.BlockSpec((blk, d), lambda i: (i, 0))


def _full_spec(shape):
    return pl.BlockSpec(shape, lambda i: tuple(0 for _ in shape))


def kernel(x, edge_index, features, W0, b0, Wl0, bl0, W1, b1, Wl1, bl1,
           order_weights, Wc, bc):
    n, fin = x.shape
    nhid = W0.shape[1]
    ncls = Wc.shape[1]
    ncpad = 48
    nc, ns = _sc_info()
    nw = nc * ns

    npad = ((n + 1 + _BLK - 1) // _BLK) * _BLK
    grid = npad // _BLK

    idt = edge_index.dtype
    loop = jnp.arange(n, dtype=idt)
    src = jnp.concatenate([edge_index[0], loop])
    dst = jnp.concatenate([edge_index[1], loop])
    et = src.shape[0]
    nch = -(-et // (nw * _CH))
    nch = -(-nch // _NBUF) * _NBUF
    epad = nw * nch * _CH
    srcp = jnp.full((epad,), n, idt).at[:et].set(src).reshape(nw, nch, _CH)
    dstp = jnp.full((epad,), n, idt).at[:et].set(dst).reshape(nw, nch, _CH)

    wl1a = order_weights[0] * Wl1[:fin]
    wl1b = order_weights[1] * Wl1[fin:]
    w1a, w1b = W1[:nhid], W1[nhid:]
    wca, wcb = Wc[:nhid], Wc[nhid:]
    b0r = b0[None, :]
    b1r = b1[None, :]
    bl0r = bl0[None, :]
    bl1r = bl1[None, :]
    bcr = bc[None, :]

    zdeg = jnp.zeros((npad,), jnp.float32)

    deg_fn = _make_deg(npad, nch, nc, ns)
    spmm_h = _make_spmm(npad, nhid, nch, nc, ns)
    spmm_c = _make_spmm(npad, ncpad, nch, nc, ns)

    degp = deg_fn(dstp, zdeg)

    y0, dinv, pre1, pre2 = pl.pallas_call(
        _tc_pre_body,
        grid=(grid,),
        in_specs=[
            pl.BlockSpec((2, _BLK), lambda i: (0, i)),
            _row_spec(fin),
            pl.BlockSpec((3, _BLK, fin), lambda i: (0, i, 0)),
            _full_spec((fin, nhid)), _full_spec((fin, nhid)),
            _full_spec((1, nhid)),
            _full_spec((fin, nhid)), _full_spec((fin, nhid)),
            _full_spec((1, nhid)),
            _full_spec((nhid, nhid)), _full_spec((nhid, ncls)),
        ],
        out_specs=[_row_spec(128), _row_spec(1),
                   _row_spec(nhid), _row_spec(ncls)],
        out_shape=[
            jax.ShapeDtypeStruct((npad, 128), jnp.float32),
            jax.ShapeDtypeStruct((npad, 1), jnp.float32),
            jax.ShapeDtypeStruct((npad, nhid), jnp.float32),
            jax.ShapeDtypeStruct((npad, ncls), jnp.float32),
        ],
    )(degp, x, features, W0, Wl0, bl0r, wl1a, wl1b, bl1r, w1a, wca)

    p0 = spmm_h(y0, srcp, dstp)

    y1 = pl.pallas_call(
        _tc_mid1_body,
        grid=(grid,),
        in_specs=[
            _row_spec(128), _row_spec(nhid), _row_spec(1),
            _full_spec((1, nhid)), _full_spec((nhid, nhid)),
        ],
        out_specs=_row_spec(128),
        out_shape=jax.ShapeDtypeStruct((npad, 128), jnp.float32),
    )(p0, pre1, dinv, b0r, w1b)

    p1 = spmm_h(y1, srcp, dstp)

    yc = pl.pallas_call(
        _tc_mid2_body,
        grid=(grid,),
        in_specs=[
            _row_spec(128), _row_spec(ncls), _row_spec(1),
            _full_spec((1, nhid)), _full_spec((nhid, ncls)),
        ],
        out_specs=_row_spec(128),
        out_shape=jax.ShapeDtypeStruct((npad, 128), jnp.float32),
    )(p1, pre2, dinv, b1r, wcb)

    pc = spmm_c(yc, srcp, dstp)

    blkp = 2000
    out0, convc = pl.pallas_call(
        _tc_post_body,
        grid=(n // blkp,),
        in_specs=[
            _row_spec(128, blkp), _row_spec(1, blkp), _full_spec((1, ncls)),
        ],
        out_specs=[_row_spec(ncls, blkp), _row_spec(ncls, blkp)],
        out_shape=[
            jax.ShapeDtypeStruct((n, ncls), jnp.float32),
            jax.ShapeDtypeStruct((n, ncls), jnp.float32),
        ],
    )(pc, dinv, bcr)

    return (out0, convc)


# speedup vs baseline: 3.1982x; 1.0007x over previous
import functools

import jax
import jax.numpy as jnp
from jax import lax
from jax.experimental import pallas as pl
from jax.experimental.pallas import tpu as pltpu
from jax.experimental.pallas import tpu_sc as plsc

_CH = 128
_NBUF = 2
_BLK = 2048


def _sc_info():
    try:
        info = plsc.get_sparse_core_info()
        return info.num_cores, info.num_subcores
    except Exception:
        return 2, 16


@functools.lru_cache(maxsize=None)
def _make_deg(npad, nch, nc, ns):
    mesh = plsc.VectorSubcoreMesh(core_axis_name="c", subcore_axis_name="s",
                                  num_cores=nc, num_subcores=ns)
    rows_per_tile = npad // ns

    def body(dst_hbm, zero_hbm, out_hbm, dstv, ones_v, acc):
        c = lax.axis_index("c")
        s = lax.axis_index("s")
        wid = s * nc + c
        pltpu.sync_copy(dst_hbm.at[wid], dstv)
        for i in range(_CH // 16):
            ones_v[pl.ds(i * 16, 16)] = jnp.full((16,), 1.0, jnp.float32)

        @pl.when(s == 0)
        def _():
            pltpu.sync_copy(zero_hbm, acc)

        plsc.subcore_barrier()

        def step(a, carry):
            pltpu.sync_copy(ones_v, acc.at[dstv.at[a]], add=True)
            return carry

        lax.fori_loop(0, nch, step, 0)
        plsc.subcore_barrier()
        lo = s * rows_per_tile
        pltpu.sync_copy(acc.at[pl.ds(lo, rows_per_tile)],
                        out_hbm.at[c].at[pl.ds(lo, rows_per_tile)])

    return pl.kernel(
        body,
        out_type=jax.ShapeDtypeStruct((nc, npad), jnp.float32),
        mesh=mesh,
        compiler_params=pltpu.CompilerParams(use_tc_tiling_on_sc=False),
        scratch_types=[
            pltpu.VMEM((nch, _CH), jnp.int32),
            pltpu.VMEM((_CH,), jnp.float32),
            pltpu.VMEM_SHARED((npad,), jnp.float32),
        ],
    )


@functools.lru_cache(maxsize=None)
def _make_spmm(npad, d, nch, nc, ns):
    mesh = plsc.VectorSubcoreMesh(core_axis_name="c", subcore_axis_name="s",
                                  num_cores=nc, num_subcores=ns)
    rows_per_tile = npad // ns
    nbuf = _NBUF

    def body(y_hbm, src_hbm, dst_hbm, zero_hbm, out_hbm,
             srcv, dstv, rows, gsems, acc, yspm):
        c = lax.axis_index("c")
        s = lax.axis_index("s")
        wid = s * nc + c
        lo = s * rows_per_tile
        pltpu.sync_copy(src_hbm.at[wid], srcv)
        pltpu.sync_copy(dst_hbm.at[wid], dstv)

        pltpu.sync_copy(zero_hbm.at[pl.ds(lo, rows_per_tile), pl.ds(0, d)],
                        acc.at[pl.ds(lo, rows_per_tile)])

        pltpu.sync_copy(y_hbm.at[pl.ds(lo, rows_per_tile), pl.ds(0, d)],
                        yspm.at[pl.ds(lo, rows_per_tile)])

        plsc.subcore_barrier()

        for b in range(nbuf - 1):
            pltpu.async_copy(yspm.at[srcv.at[b]], rows[b], gsems[b])

        def outer(g2, carry):
            g = g2 * nbuf
            for b in range(nbuf):
                a = g + b
                pltpu.make_async_copy(yspm.at[srcv.at[a]], rows[b],
                                      gsems[b]).wait()

                nb = (b + nbuf - 1) % nbuf

                @pl.when(a + nbuf - 1 < nch)
                def _():
                    pltpu.async_copy(yspm.at[srcv.at[a + nbuf - 1]],
                                     rows[nb], gsems[nb])

                pltpu.sync_copy(rows[b], acc.at[dstv.at[a]], add=True)
            return carry

        lax.fori_loop(0, nch // nbuf, outer, 0)
        plsc.subcore_barrier()
        pltpu.sync_copy(acc.at[pl.ds(lo, rows_per_tile)],
                        out_hbm.at[pl.ds(lo, rows_per_tile),
                                   pl.ds(c * 64, d)])

    return pl.kernel(
        body,
        out_type=jax.ShapeDtypeStruct((npad, 128), jnp.float32),
        mesh=mesh,
        compiler_params=pltpu.CompilerParams(use_tc_tiling_on_sc=False),
        scratch_types=[
            pltpu.VMEM((nch, _CH), jnp.int32),
            pltpu.VMEM((nch, _CH), jnp.int32),
            [pltpu.VMEM((_CH, d), jnp.float32) for _ in range(nbuf)],
            [pltpu.SemaphoreType.DMA for _ in range(nbuf)],
            pltpu.VMEM_SHARED((npad, d), jnp.float32),
            pltpu.VMEM_SHARED((npad, d), jnp.float32),
        ],
    )


def _pad128(v):
    return jnp.concatenate(
        [v, jnp.zeros((v.shape[0], 128 - v.shape[1]), jnp.float32)], axis=1)


def _tc_pre_body(degp, xb, fb, w0, wl0, bl0, wl1a, wl1b, bl1, w1a, wca,
                 y0, dinvb, pre1, pre2):
    lin0 = jnp.maximum(
        jnp.dot(fb[0], wl0[...], preferred_element_type=jnp.float32)
        + bl0[...], 0.0)
    lin1 = jnp.maximum(
        jnp.dot(fb[1], wl1a[...], preferred_element_type=jnp.float32)
        + jnp.dot(fb[2], wl1b[...], preferred_element_type=jnp.float32)
        + bl1[...], 0.0)
    xw0 = jnp.dot(xb[...], w0[...], preferred_element_type=jnp.float32)
    pre1[...] = jnp.dot(lin0, w1a[...], preferred_element_type=jnp.float32)
    pre2[...] = jnp.dot(lin1, wca[...], preferred_element_type=jnp.float32)
    deg = degp[0, :] + degp[1, :]
    dinv = lax.rsqrt(jnp.maximum(deg, 1.0))[:, None]
    dinvb[...] = dinv
    y0[...] = _pad128(xw0 * dinv)


def _tc_mid1_body(p0, pre1, dinvb, b0, w1b, y1):
    dinv = dinvb[...]
    p = p0[...]
    conv0 = jnp.maximum(dinv * (p[:, :64] + p[:, 64:]) + b0[...], 0.0)
    y1[...] = _pad128(
        (pre1[...] + jnp.dot(conv0, w1b[...],
                             preferred_element_type=jnp.float32)) * dinv)


def _tc_mid2_body(p1, pre2, dinvb, b1, wcb, yc):
    dinv = dinvb[...]
    p = p1[...]
    conv1 = jnp.maximum(dinv * (p[:, :64] + p[:, 64:]) + b1[...], 0.0)
    v = (pre2[...] + jnp.dot(conv1, wcb[...],
                             preferred_element_type=jnp.float32)) * dinv
    yc[...] = _pad128(v)


def _tc_post_body(pc, dinvb, bc, out0, convc):
    ncls = bc.shape[1]
    p = pc[...]
    v = dinvb[...] * (p[:, :ncls] + p[:, 64:64 + ncls]) + bc[...]
    m = jnp.max(v, axis=-1, keepdims=True)
    lse = jnp.log(jnp.sum(jnp.exp(v - m), axis=-1, keepdims=True)) + m
    convc[...] = v
    out0[...] = v - lse


def _row_spec(d, blk=_BLK):
    return pl.BlockSpec((blk, d), lambda i: (i, 0))


def _full_spec(shape):
    return pl.BlockSpec(shape, lambda i: tuple(0 for _ in shape))


def kernel(x, edge_index, features, W0, b0, Wl0, bl0, W1, b1, Wl1, bl1,
           order_weights, Wc, bc):
    n, fin = x.shape
    nhid = W0.shape[1]
    ncls = Wc.shape[1]
    nc, ns = _sc_info()
    nw = nc * ns

    npad = ((n + 1 + _BLK - 1) // _BLK) * _BLK
    grid = npad // _BLK

    idt = edge_index.dtype
    loop = jnp.arange(n, dtype=idt)
    src = jnp.concatenate([edge_index[0], loop])
    dst = jnp.concatenate([edge_index[1], loop])
    et = src.shape[0]
    nch = -(-et // (nw * _CH))
    nch = -(-nch // _NBUF) * _NBUF
    epad = nw * nch * _CH
    srcp = jnp.full((epad,), n, idt).at[:et].set(src).reshape(nw, nch, _CH)
    dstp = jnp.full((epad,), n, idt).at[:et].set(dst).reshape(nw, nch, _CH)

    wl1a = order_weights[0] * Wl1[:fin]
    wl1b = order_weights[1] * Wl1[fin:]
    w1a, w1b = W1[:nhid], W1[nhid:]
    wca, wcb = Wc[:nhid], Wc[nhid:]
    b0r = b0[None, :]
    b1r = b1[None, :]
    bl0r = bl0[None, :]
    bl1r = bl1[None, :]
    bcr = bc[None, :]

    zdeg = jnp.zeros((npad,), jnp.float32)
    z128 = jnp.zeros((npad, 128), jnp.float32)

    deg_fn = _make_deg(npad, nch, nc, ns)
    spmm_h = _make_spmm(npad, nhid, nch, nc, ns)
    spmm_c = _make_spmm(npad, ncls, nch, nc, ns)

    degp = deg_fn(dstp, zdeg)

    y0, dinv, pre1, pre2 = pl.pallas_call(
        _tc_pre_body,
        grid=(grid,),
        in_specs=[
            pl.BlockSpec((2, _BLK), lambda i: (0, i)),
            _row_spec(fin),
            pl.BlockSpec((3, _BLK, fin), lambda i: (0, i, 0)),
            _full_spec((fin, nhid)), _full_spec((fin, nhid)),
            _full_spec((1, nhid)),
            _full_spec((fin, nhid)), _full_spec((fin, nhid)),
            _full_spec((1, nhid)),
            _full_spec((nhid, nhid)), _full_spec((nhid, ncls)),
        ],
        out_specs=[_row_spec(128), _row_spec(1),
                   _row_spec(nhid), _row_spec(ncls)],
        out_shape=[
            jax.ShapeDtypeStruct((npad, 128), jnp.float32),
            jax.ShapeDtypeStruct((npad, 1), jnp.float32),
            jax.ShapeDtypeStruct((npad, nhid), jnp.float32),
            jax.ShapeDtypeStruct((npad, ncls), jnp.float32),
        ],
    )(degp, x, features, W0, Wl0, bl0r, wl1a, wl1b, bl1r, w1a, wca)

    p0 = spmm_h(y0, srcp, dstp, z128)

    y1 = pl.pallas_call(
        _tc_mid1_body,
        grid=(grid,),
        in_specs=[
            _row_spec(128), _row_spec(nhid), _row_spec(1),
            _full_spec((1, nhid)), _full_spec((nhid, nhid)),
        ],
        out_specs=_row_spec(128),
        out_shape=jax.ShapeDtypeStruct((npad, 128), jnp.float32),
    )(p0, pre1, dinv, b0r, w1b)

    p1 = spmm_h(y1, srcp, dstp, z128)

    yc = pl.pallas_call(
        _tc_mid2_body,
        grid=(grid,),
        in_specs=[
            _row_spec(128), _row_spec(ncls), _row_spec(1),
            _full_spec((1, nhid)), _full_spec((nhid, ncls)),
        ],
        out_specs=_row_spec(128),
        out_shape=jax.ShapeDtypeStruct((npad, 128), jnp.float32),
    )(p1, pre2, dinv, b1r, wcb)

    pc = spmm_c(yc, srcp, dstp, z128)

    blkp = 2000
    out0, convc = pl.pallas_call(
        _tc_post_body,
        grid=(n // blkp,),
        in_specs=[
            _row_spec(128, blkp), _row_spec(1, blkp), _full_spec((1, ncls)),
        ],
        out_specs=[_row_spec(ncls, blkp), _row_spec(ncls, blkp)],
        out_shape=[
            jax.ShapeDtypeStruct((n, ncls), jnp.float32),
            jax.ShapeDtypeStruct((n, ncls), jnp.float32),
        ],
    )(pc, dinv, bcr)

    return (out0, convc)
